# baseline (speedup 1.0000x reference)
import functools

import jax
import jax.numpy as jnp
from jax import lax
from jax.experimental import pallas as pl
from jax.experimental.pallas import tpu as pltpu
from jax.experimental.pallas import tpu_sc as plsc

B, H, D = 16384, 200, 64
NC, NS = 2, 16
NW = NC * NS
BT = 128
NBT = B // BT
BT_W = NBT // NW
HPAD = H + 2

_mesh = plsc.VectorSubcoreMesh(core_axis_name="c", subcore_axis_name="s")


@functools.partial(
    pl.kernel,
    mesh=_mesh,
    out_type=jax.ShapeDtypeStruct((H, D // 8, NBT, 8 * BT), jnp.float32),
    scratch_types=[
        pltpu.VMEM((BT, H), jnp.int32),
        pltpu.VMEM((HPAD, BT), jnp.int32),
        pltpu.VMEM((BT, D), jnp.float32),
        pltpu.VMEM((BT, D), jnp.float32),
        pltpu.VMEM((D // 8, 8 * BT), jnp.float32),
        pltpu.VMEM((D // 8, 8 * BT), jnp.float32),
        pltpu.SemaphoreType.DMA,
        pltpu.SemaphoreType.DMA,
        pltpu.SemaphoreType.DMA,
        pltpu.SemaphoreType.DMA,
    ],
    compiler_params=pltpu.CompilerParams(
        use_tc_tiling_on_sc=False,
        needs_layout_passes=False,
        disable_bounds_checks=True,
    ),
)
def _embed(x_hbm, table_hbm, out_hbm, idx_raw, idx_t, rows0, rows1,
           trows0, trows1, gsem0, gsem1, osem0, osem1):
    rows_v = [rows0, rows1]
    trows_v = [trows0, trows1]
    gsems = [gsem0, gsem1]
    osems = [osem0, osem1]

    wid = lax.axis_index("s") * NC + lax.axis_index("c")
    iota = lax.iota(jnp.int32, 16)
    lane_sb = [iota + sb * 16 for sb in range(8)]
    zeros16 = jnp.zeros((16,), jnp.int32)

    def bt_body(k, carry):
        bt = wid * BT_W + k
        b0 = bt * BT

        pltpu.sync_copy(x_hbm.at[pl.ds(b0, BT)], idx_raw)

        def idx_body(h, c):
            hvec = jnp.broadcast_to(h, (16,)).astype(jnp.int32)
            for sb in range(8):
                v = plsc.load_gather(idx_raw, [lane_sb[sb], hvec])
                idx_t[h, pl.ds(sb * 16, 16)] = v
            return c

        lax.fori_loop(0, H, idx_body, 0)
        for sb in range(8):
            idx_t[H, pl.ds(sb * 16, 16)] = zeros16
            idx_t[H + 1, pl.ds(sb * 16, 16)] = zeros16

        def fire_gather(h, p):
            pltpu.async_copy(
                table_hbm.at[idx_t.at[h]], rows_v[p], gsems[p]
            )

        def wait_gather(p):
            pltpu.make_async_copy(
                table_hbm.at[pl.ds(0, BT)], rows_v[p], gsems[p]
            ).wait()

        def transpose(p):
            rows = rows_v[p]
            trows = trows_v[p]
            for d in range(D):
                dvec = jnp.full((16,), d, jnp.int32)
                dt, off = d // 8, (d % 8) * BT
                for sb in range(8):
                    v = plsc.load_gather(rows, [lane_sb[sb], dvec])
                    trows[dt, pl.ds(off + sb * 16, 16)] = v

        def fire_store(h, p):
            pltpu.async_copy(trows_v[p], out_hbm.at[h, :, bt], osems[p])

        def wait_store(p):
            pltpu.make_async_copy(
                trows_v[p], out_hbm.at[0, :, 0], osems[p]
            ).wait()

        fire_gather(0, 0)
        fire_gather(1, 1)
        for p in range(2):
            wait_gather(p)
            transpose(p)
            fire_store(p, p)
        fire_gather(2, 0)
        fire_gather(3, 1)

        def h_body(i, c):
            for p in range(2):
                h = 2 * i + p
                wait_gather(p)
                wait_store(p)
                transpose(p)
                fire_store(h, p)
                fire_gather(h + 2, p)
            return c

        lax.fori_loop(1, H // 2, h_body, 0)

        for p in range(2):
            wait_gather(p)
            wait_store(p)
        return carry

    lax.fori_loop(0, BT_W, bt_body, 0)


def kernel(x, weight):
    out4 = _embed(x, weight)
    r5 = out4.reshape(H, D // 8, NBT, 8, BT)
    return jnp.transpose(r5, (2, 4, 0, 1, 3)).reshape(B, H, D)


# speedup vs baseline: 2.1995x; 2.1995x over previous
import functools

import jax
import jax.numpy as jnp
from jax import lax
from jax.experimental import pallas as pl
from jax.experimental.pallas import tpu as pltpu
from jax.experimental.pallas import tpu_sc as plsc

B, H, D = 16384, 200, 64
NC, NS = 2, 16
NW = NC * NS
BT = 128
NBT = B // BT
BT_W = NBT // NW
HPAD = H + 2
TP = BT + 1

_mesh = plsc.VectorSubcoreMesh(core_axis_name="c", subcore_axis_name="s")


@functools.partial(
    pl.kernel,
    mesh=_mesh,
    out_type=jax.ShapeDtypeStruct((H, D // 8, NBT, 8, BT), jnp.float32),
    scratch_types=[
        pltpu.VMEM((BT, H), jnp.int32),
        pltpu.VMEM((HPAD, BT), jnp.int32),
        pltpu.VMEM((BT, D), jnp.float32),
        pltpu.VMEM((BT, D), jnp.float32),
        pltpu.VMEM((D, TP), jnp.float32),
        pltpu.VMEM((D, TP), jnp.float32),
        pltpu.SemaphoreType.DMA,
        pltpu.SemaphoreType.DMA,
        pltpu.SemaphoreType.DMA,
        pltpu.SemaphoreType.DMA,
    ],
    compiler_params=pltpu.CompilerParams(
        use_tc_tiling_on_sc=False,
        needs_layout_passes=False,
        disable_bounds_checks=True,
    ),
)
def _embed(x_hbm, table_hbm, out_hbm, idx_raw, idx_t, rows0, rows1,
           trows0, trows1, gsem0, gsem1, osem0, osem1):
    rows_v = [rows0, rows1]
    trows_v = [trows0, trows1]
    gsems = [gsem0, gsem1]
    osems = [osem0, osem1]

    wid = lax.axis_index("s") * NC + lax.axis_index("c")
    iota = lax.iota(jnp.int32, 16)
    lane_sb = [iota + sb * 16 for sb in range(8)]
    row_dg = [iota + dg * 16 for dg in range(4)]
    zeros16 = jnp.zeros((16,), jnp.int32)

    def bt_body(k, carry):
        bt = wid * BT_W + k
        b0 = bt * BT

        pltpu.sync_copy(x_hbm.at[pl.ds(b0, BT)], idx_raw)

        def idx_body(h, c):
            hvec = jnp.broadcast_to(h, (16,)).astype(jnp.int32)
            for sb in range(8):
                v = plsc.load_gather(idx_raw, [lane_sb[sb], hvec])
                idx_t[h, pl.ds(sb * 16, 16)] = v
            return c

        lax.fori_loop(0, H, idx_body, 0)
        for sb in range(8):
            idx_t[H, pl.ds(sb * 16, 16)] = zeros16
            idx_t[H + 1, pl.ds(sb * 16, 16)] = zeros16

        def fire_gather(h, p):
            pltpu.async_copy(
                table_hbm.at[idx_t.at[h]], rows_v[p], gsems[p]
            )

        def wait_gather(p):
            pltpu.make_async_copy(
                table_hbm.at[pl.ds(0, BT)], rows_v[p], gsems[p]
            ).wait()

        def transpose(p):
            rows = rows_v[p]
            trows = trows_v[p]
            for b in range(BT):
                bvec = jnp.full((16,), b, jnp.int32)
                for dg in range(4):
                    v = rows[b, pl.ds(dg * 16, 16)]
                    plsc.store_scatter(trows, [row_dg[dg], bvec], v)

        def fire_store(h, p):
            for dt in range(D // 8):
                pltpu.async_copy(
                    trows_v[p].at[pl.ds(dt * 8, 8), pl.ds(0, BT)],
                    out_hbm.at[h, dt, bt],
                    osems[p],
                )

        def wait_store(p):
            for dt in range(D // 8):
                pltpu.make_async_copy(
                    trows_v[p].at[pl.ds(dt * 8, 8), pl.ds(0, BT)],
                    out_hbm.at[0, dt, 0],
                    osems[p],
                ).wait()

        fire_gather(0, 0)
        fire_gather(1, 1)

        def h_body(i, c):
            for p in range(2):
                h = 2 * i + p
                wait_gather(p)

                @pl.when(i > 0)
                def _():
                    wait_store(p)

                transpose(p)
                fire_store(h, p)
                fire_gather(h + 2, p)
            return c

        lax.fori_loop(0, H // 2, h_body, 0)

        for p in range(2):
            wait_gather(p)
            wait_store(p)
        return carry

    lax.fori_loop(0, BT_W, bt_body, 0)


def kernel(x, weight):
    out5 = _embed(x, weight)
    return jnp.transpose(out5, (2, 4, 0, 1, 3)).reshape(B, H, D)
